# baseline (speedup 1.0000x reference)
import functools

import jax
import jax.numpy as jnp
from jax import lax
from jax.experimental import pallas as pl
from jax.experimental.pallas import tpu as pltpu
from jax.experimental.pallas import tpu_sc as plsc

_VOCAB = 100000
_N = 4096
_K = 25
_KP = 32
_DX = 64
_DY = 64
_LO = -4.0
_HI = 4.0
_STEP = (_HI - _LO) / (_DX - 1)

_NB = 8
_CW = _NB * _KP
_RW = _NB * _DX


_GP = 8


def _field_body(mux_ref, muy_ref, w_ref, out_ref):
    gx = lax.broadcasted_iota(jnp.int32, (_DX, 1), 0).astype(jnp.float32) * _STEP + _LO
    gy = lax.broadcasted_iota(jnp.int32, (1, _DY), 1).astype(jnp.float32) * _STEP + _LO
    r_i = lax.broadcasted_iota(jnp.int32, (_RW, 1), 0)
    c_i = lax.broadcasted_iota(jnp.int32, (1, _CW), 1)
    mask = (r_i // _DX) == (c_i // _KP)
    k_i = lax.broadcasted_iota(jnp.int32, (_CW, 1), 0)
    kmask = (k_i % _KP) < _K
    for g in range(_GP):
        mux = mux_ref[g]
        muy = muy_ref[g]
        wv = jnp.where(kmask, w_ref[g], 0.0)
        ax = jnp.exp(-0.5 * (gx - mux) ** 2)
        tiled = jnp.concatenate([ax] * _NB, axis=0)
        a_bd = jnp.where(mask, tiled, 0.0).astype(jnp.bfloat16)
        b = (jnp.exp(-0.5 * (gy - muy) ** 2) * wv).astype(jnp.bfloat16)
        out_ref[pl.ds(g * _RW, _RW), :] = jnp.dot(
            a_bd, b, preferred_element_type=jnp.float32)


def _field(mux_p, muy_p, w_p, interpret=False):
    grid = _N // (_NB * _GP)
    return pl.pallas_call(
        _field_body,
        grid=(grid,),
        in_specs=[
            pl.BlockSpec((_GP, 1, _CW), lambda i: (i, 0, 0)),
            pl.BlockSpec((_GP, _CW, 1), lambda i: (i, 0, 0)),
            pl.BlockSpec((_GP, _CW, 1), lambda i: (i, 0, 0)),
        ],
        out_specs=pl.BlockSpec((_GP * _RW, _DY), lambda i: (i, 0)),
        out_shape=jax.ShapeDtypeStruct((_N * _DX, _DY), jnp.float32),
        interpret=interpret,
    )(mux_p, muy_p, w_p)


_TPW = _N // 32


def _sc_gather(ranks, mu2, w):
    mesh = plsc.VectorSubcoreMesh(core_axis_name="c", subcore_axis_name="s")

    nch = _TPW * _KP // 128

    @functools.partial(
        pl.kernel,
        mesh=mesh,
        out_type=[jax.ShapeDtypeStruct((_N * _KP // 128, 128), jnp.float32)] * 3,
        scratch_types=[
            pltpu.VMEM((nch, 128), jnp.int32),
            pltpu.VMEM((nch, 128), jnp.int32),
            pltpu.VMEM((nch, 128), jnp.int32),
            pltpu.VMEM((nch, 128), jnp.float32),
            pltpu.VMEM((nch, 128), jnp.float32),
            pltpu.VMEM((nch, 128), jnp.float32),
            pltpu.SemaphoreType.DMA,
            pltpu.SemaphoreType.DMA,
            pltpu.SemaphoreType.DMA,
        ],
    )
    def k(idxx_h, idxy_h, idxw_h, mu_h, w_h, mux_h, muy_h, wp_h,
          idxx, idxy, idxw, dmux, dmuy, dw, sem1, sem2, sem3):
        wid = lax.axis_index("s") * 2 + lax.axis_index("c")
        row0 = wid * nch
        pltpu.sync_copy(idxx_h.at[pl.ds(row0, nch)], idxx)
        pltpu.sync_copy(idxy_h.at[pl.ds(row0, nch)], idxy)
        pltpu.sync_copy(idxw_h.at[pl.ds(row0, nch)], idxw)

        def body(i, carry):
            cpx = pltpu.async_copy(mu_h.at[idxx.at[i]], dmux.at[i], sem1)
            cpy = pltpu.async_copy(mu_h.at[idxy.at[i]], dmuy.at[i], sem2)
            cpw = pltpu.async_copy(w_h.at[idxw.at[i]], dw.at[i], sem3)
            cpx.wait()
            cpy.wait()
            cpw.wait()
            return carry

        lax.fori_loop(0, nch, body, 0)
        pltpu.sync_copy(dmux, mux_h.at[pl.ds(row0, nch)])
        pltpu.sync_copy(dmuy, muy_h.at[pl.ds(row0, nch)])
        pltpu.sync_copy(dw, wp_h.at[pl.ds(row0, nch)])

    kp_col = jnp.minimum(jnp.arange(_KP, dtype=jnp.int32), _K - 1)
    rb = ranks.astype(jnp.int32)[:, None]
    idxx_a = (rb * (2 * _K) + 2 * kp_col).reshape(_N * _KP // 128, 128)
    idxy_a = (rb * (2 * _K) + 2 * kp_col + 1).reshape(_N * _KP // 128, 128)
    idxw_a = (rb * _K + kp_col).reshape(_N * _KP // 128, 128)
    return k(idxx_a, idxy_a, idxw_a, mu2, w)


def kernel(ranks, mu, w):
    muf = mu.reshape(_VOCAB * 2 * _K)
    wf = w.reshape(_VOCAB * _K)
    mux_p, muy_p, w_p = _sc_gather(ranks, muf, wf)

    g = _N // _NB
    out = _field(
        mux_p.reshape(g, 1, _CW),
        muy_p.reshape(g, _CW, 1),
        w_p.reshape(g, _CW, 1),
    )
    return out.reshape(_N, _DX * _DY)


# speedup vs baseline: 1.0036x; 1.0036x over previous
import functools

import jax
import jax.numpy as jnp
from jax import lax
from jax.experimental import pallas as pl
from jax.experimental.pallas import tpu as pltpu
from jax.experimental.pallas import tpu_sc as plsc

_VOCAB = 100000
_N = 4096
_K = 25
_KP = 32
_DX = 64
_DY = 64
_LO = -4.0
_HI = 4.0
_STEP = (_HI - _LO) / (_DX - 1)

_NB = 8
_CW = _NB * _KP
_RW = _NB * _DX


_GP = 8


def _field_body(mux_ref, muy_ref, w_ref, out_ref):
    gx = lax.broadcasted_iota(jnp.int32, (_DX, 1), 0).astype(jnp.float32) * _STEP + _LO
    gy = lax.broadcasted_iota(jnp.int32, (1, _DY), 1).astype(jnp.float32) * _STEP + _LO
    r_i = lax.broadcasted_iota(jnp.int32, (_RW, 1), 0)
    c_i = lax.broadcasted_iota(jnp.int32, (1, _CW), 1)
    mask = (r_i // _DX) == (c_i // _KP)
    k_i = lax.broadcasted_iota(jnp.int32, (_CW, 1), 0)
    kmask = (k_i % _KP) < _K
    for g in range(_GP):
        mux = mux_ref[g]
        muy = muy_ref[g]
        wv = jnp.where(kmask, w_ref[g], 0.0)
        ax = jnp.exp(-0.5 * (gx - mux) ** 2)
        tiled = jnp.concatenate([ax] * _NB, axis=0)
        a_bd = jnp.where(mask, tiled, 0.0).astype(jnp.bfloat16)
        b = (jnp.exp(-0.5 * (gy - muy) ** 2) * wv).astype(jnp.bfloat16)
        out_ref[pl.ds(g * _RW, _RW), :] = jnp.dot(
            a_bd, b, preferred_element_type=jnp.float32)


def _field(mux_p, muy_p, w_p, interpret=False):
    grid = _N // (_NB * _GP)
    return pl.pallas_call(
        _field_body,
        grid=(grid,),
        in_specs=[
            pl.BlockSpec((_GP, 1, _CW), lambda i: (i, 0, 0)),
            pl.BlockSpec((_GP, _CW, 1), lambda i: (i, 0, 0)),
            pl.BlockSpec((_GP, _CW, 1), lambda i: (i, 0, 0)),
        ],
        out_specs=pl.BlockSpec((_GP * _RW, _DY), lambda i: (i, 0)),
        out_shape=jax.ShapeDtypeStruct((_N * _DX, _DY), jnp.float32),
        interpret=interpret,
    )(mux_p, muy_p, w_p)


_TPW = _N // 32


def _sc_gather(ranks, mu2, w):
    mesh = plsc.VectorSubcoreMesh(core_axis_name="c", subcore_axis_name="s")

    nch = _TPW * _KP // 128

    @functools.partial(
        pl.kernel,
        mesh=mesh,
        out_type=[jax.ShapeDtypeStruct((_N * _KP // 128, 128), jnp.float32)] * 3,
        scratch_types=[
            pltpu.VMEM((nch, 128), jnp.int32),
            pltpu.VMEM((nch, 128), jnp.int32),
            pltpu.VMEM((nch, 128), jnp.int32),
            pltpu.VMEM((nch, 128), jnp.float32),
            pltpu.VMEM((nch, 128), jnp.float32),
            pltpu.VMEM((nch, 128), jnp.float32),
            pltpu.SemaphoreType.DMA,
            pltpu.SemaphoreType.DMA,
            pltpu.SemaphoreType.DMA,
        ],
    )
    def k(idxx_h, idxy_h, idxw_h, mu_h, w_h, mux_h, muy_h, wp_h,
          idxx, idxy, idxw, dmux, dmuy, dw, sem1, sem2, sem3):
        wid = lax.axis_index("s") * 2 + lax.axis_index("c")
        row0 = wid * nch
        pltpu.sync_copy(idxx_h.at[pl.ds(row0, nch)], idxx)
        pltpu.sync_copy(idxy_h.at[pl.ds(row0, nch)], idxy)
        pltpu.sync_copy(idxw_h.at[pl.ds(row0, nch)], idxw)

        def body(i, carry):
            pltpu.async_copy(mu_h.at[idxx.at[i]], dmux.at[i], sem1)
            pltpu.async_copy(mu_h.at[idxy.at[i]], dmuy.at[i], sem2)
            pltpu.async_copy(w_h.at[idxw.at[i]], dw.at[i], sem3)
            return carry

        lax.fori_loop(0, nch, body, 0)
        pltpu.make_async_copy(mux_h.at[pl.ds(row0, nch)], dmux, sem1).wait()
        pltpu.make_async_copy(muy_h.at[pl.ds(row0, nch)], dmuy, sem2).wait()
        pltpu.make_async_copy(wp_h.at[pl.ds(row0, nch)], dw, sem3).wait()
        pltpu.sync_copy(dmux, mux_h.at[pl.ds(row0, nch)])
        pltpu.sync_copy(dmuy, muy_h.at[pl.ds(row0, nch)])
        pltpu.sync_copy(dw, wp_h.at[pl.ds(row0, nch)])

    kp_col = jnp.minimum(jnp.arange(_KP, dtype=jnp.int32), _K - 1)
    rb = ranks.astype(jnp.int32)[:, None]
    idxx_a = (rb * (2 * _K) + 2 * kp_col).reshape(_N * _KP // 128, 128)
    idxy_a = (rb * (2 * _K) + 2 * kp_col + 1).reshape(_N * _KP // 128, 128)
    idxw_a = (rb * _K + kp_col).reshape(_N * _KP // 128, 128)
    return k(idxx_a, idxy_a, idxw_a, mu2, w)


def kernel(ranks, mu, w):
    muf = mu.reshape(_VOCAB * 2 * _K)
    wf = w.reshape(_VOCAB * _K)
    mux_p, muy_p, w_p = _sc_gather(ranks, muf, wf)

    g = _N // _NB
    out = _field(
        mux_p.reshape(g, 1, _CW),
        muy_p.reshape(g, _CW, 1),
        w_p.reshape(g, _CW, 1),
    )
    return out.reshape(_N, _DX * _DY)
